# baseline (speedup 1.0000x reference)
import jax
import jax.numpy as jnp
from jax import lax
from jax.experimental import pallas as pl
from jax.experimental.pallas import tpu as pltpu
from jax.experimental.pallas import tpu_sc as plsc

_W = 0.5
_N = 10000
_D = 128
_E = 320000
_NC = 2
_NS = 16
_NW = _NC * _NS
_CHUNK = 128
_CPW = 80
_EPAD = _NW * _CPW * _CHUNK
_ACC_ROWS = 10240
_RPS = _ACC_ROWS // _NS


def _agg_body(x_hbm, src_hbm, dst_hbm, ones_hbm, zra_hbm, zrc_hbm,
              sum_hbm, cnt_hbm,
              src_a, dst_a, src_b, dst_b, rows_a, rows_b, ones_v, zc_v,
              acc_sh, cnt_sh, sem_a, sem_b, sem_ia, sem_ib, sem_p,
              sem_sa, sem_sb, sem_ca, sem_cb):
    c = lax.axis_index("c")
    s = lax.axis_index("s")
    w = s * _NC + c

    pltpu.sync_copy(zra_hbm, rows_a)
    pltpu.sync_copy(zrc_hbm, zc_v)
    pltpu.sync_copy(ones_hbm, ones_v)
    r0 = s * _RPS
    for k in range(5):
        pltpu.async_copy(rows_a, acc_sh.at[pl.ds(r0 + k * 128, 128)], sem_p)
        pltpu.async_copy(zc_v, cnt_sh.at[pl.ds(r0 + k * 128, 128)], sem_p)
    for k in range(5):
        pltpu.make_async_copy(rows_a, acc_sh.at[pl.ds(r0, 128)], sem_p).wait()
        pltpu.make_async_copy(zc_v, cnt_sh.at[pl.ds(r0, 128)], sem_p).wait()
    plsc.subcore_barrier()

    def gather_wait(src_c, rows_v, sem):
        pltpu.make_async_copy(x_hbm.at[src_c], rows_v, sem).wait()

    def scatter_start(rows_v, dst_c, sem_s, sem_c):
        pltpu.async_copy(rows_v, acc_sh.at[dst_c], sem_s, add=True)
        pltpu.async_copy(ones_v, cnt_sh.at[dst_c], sem_c, add=True)

    def scatter_wait(rows_v, dst_c, sem_s, sem_c):
        pltpu.make_async_copy(rows_v, acc_sh.at[dst_c], sem_s).wait()
        pltpu.make_async_copy(ones_v, cnt_sh.at[dst_c], sem_c).wait()

    pltpu.async_copy(src_hbm.at[w, 0], src_a, sem_ia)
    pltpu.async_copy(dst_hbm.at[w, 0], dst_a, sem_ia)
    pltpu.async_copy(src_hbm.at[w, 1], src_b, sem_ib)
    pltpu.async_copy(dst_hbm.at[w, 1], dst_b, sem_ib)
    pltpu.make_async_copy(src_hbm.at[w, 0], src_a, sem_ia).wait()
    pltpu.make_async_copy(dst_hbm.at[w, 0], dst_a, sem_ia).wait()
    pltpu.async_copy(x_hbm.at[src_a], rows_a, sem_a)
    pltpu.make_async_copy(src_hbm.at[w, 1], src_b, sem_ib).wait()
    pltpu.make_async_copy(dst_hbm.at[w, 1], dst_b, sem_ib).wait()
    pltpu.async_copy(x_hbm.at[src_b], rows_b, sem_b)

    def body(t, carry):
        i = 2 * t
        gather_wait(src_a, rows_a, sem_a)
        scatter_start(rows_a, dst_a, sem_sa, sem_ca)
        pltpu.async_copy(src_hbm.at[w, i + 2], src_a, sem_ia)
        gather_wait(src_b, rows_b, sem_b)
        scatter_start(rows_b, dst_b, sem_sb, sem_cb)
        pltpu.async_copy(src_hbm.at[w, i + 3], src_b, sem_ib)
        scatter_wait(rows_a, dst_a, sem_sa, sem_ca)
        pltpu.async_copy(dst_hbm.at[w, i + 2], dst_a, sem_ia)
        pltpu.make_async_copy(src_hbm.at[w, 0], src_a, sem_ia).wait()
        pltpu.make_async_copy(dst_hbm.at[w, 0], dst_a, sem_ia).wait()
        pltpu.async_copy(x_hbm.at[src_a], rows_a, sem_a)
        scatter_wait(rows_b, dst_b, sem_sb, sem_cb)
        pltpu.async_copy(dst_hbm.at[w, i + 3], dst_b, sem_ib)
        pltpu.make_async_copy(src_hbm.at[w, 0], src_b, sem_ib).wait()
        pltpu.make_async_copy(dst_hbm.at[w, 0], dst_b, sem_ib).wait()
        pltpu.async_copy(x_hbm.at[src_b], rows_b, sem_b)
        return carry
    lax.fori_loop(0, _CPW // 2 - 1, body, 0)

    gather_wait(src_a, rows_a, sem_a)
    scatter_start(rows_a, dst_a, sem_sa, sem_ca)
    gather_wait(src_b, rows_b, sem_b)
    scatter_start(rows_b, dst_b, sem_sb, sem_cb)
    scatter_wait(rows_a, dst_a, sem_sa, sem_ca)
    scatter_wait(rows_b, dst_b, sem_sb, sem_cb)

    plsc.subcore_barrier()

    for k in range(5):
        pltpu.async_copy(acc_sh.at[pl.ds(r0 + k * 128, 128)],
                         sum_hbm.at[c, pl.ds(r0 + k * 128, 128)], sem_p)
        pltpu.async_copy(cnt_sh.at[pl.ds(r0 + k * 128, 128)],
                         cnt_hbm.at[c, pl.ds(r0 + k * 128, 128)], sem_p)
    for k in range(5):
        pltpu.make_async_copy(acc_sh.at[pl.ds(r0, 128)],
                              sum_hbm.at[c, pl.ds(r0, 128)], sem_p).wait()
        pltpu.make_async_copy(cnt_sh.at[pl.ds(r0, 128)],
                              cnt_hbm.at[c, pl.ds(r0, 128)], sem_p).wait()


_agg = pl.kernel(
    _agg_body,
    mesh=plsc.VectorSubcoreMesh(core_axis_name="c", subcore_axis_name="s"),
    out_type=[
        jax.ShapeDtypeStruct((_NC, _ACC_ROWS, _D), jnp.float32),
        jax.ShapeDtypeStruct((_NC, _ACC_ROWS), jnp.float32),
    ],
    scratch_types=[
        pltpu.VMEM((_CHUNK,), jnp.int32),
        pltpu.VMEM((_CHUNK,), jnp.int32),
        pltpu.VMEM((_CHUNK,), jnp.int32),
        pltpu.VMEM((_CHUNK,), jnp.int32),
        pltpu.VMEM((_CHUNK, _D), jnp.float32),
        pltpu.VMEM((_CHUNK, _D), jnp.float32),
        pltpu.VMEM((_CHUNK,), jnp.float32),
        pltpu.VMEM((_CHUNK,), jnp.float32),
        pltpu.VMEM_SHARED((_ACC_ROWS, _D), jnp.float32),
        pltpu.VMEM_SHARED((_ACC_ROWS,), jnp.float32),
        pltpu.SemaphoreType.DMA,
        pltpu.SemaphoreType.DMA,
        pltpu.SemaphoreType.DMA,
        pltpu.SemaphoreType.DMA,
        pltpu.SemaphoreType.DMA,
        pltpu.SemaphoreType.DMA,
        pltpu.SemaphoreType.DMA,
        pltpu.SemaphoreType.DMA,
        pltpu.SemaphoreType.DMA,
    ],
)


def _epi_body(x_ref, s_ref, c_ref, o_ref):
    cnt = c_ref[0, 0:_N] + c_ref[1, 0:_N]
    cnt = jnp.maximum(cnt, 1.0).reshape(_N, 1)
    mean = (s_ref[0, 0:_N] + s_ref[1, 0:_N]) / cnt
    o_ref[...] = x_ref[...] + _W * mean


_epi = pl.pallas_call(
    _epi_body,
    out_shape=jax.ShapeDtypeStruct((_N, _D), jnp.float32),
)


def kernel(x, edge_index):
    src = edge_index[0].astype(jnp.int32)
    dst = edge_index[1].astype(jnp.int32)
    pad = _EPAD - _E
    r = jnp.arange(pad, dtype=jnp.int32)
    src = jnp.concatenate([src, r % _N]).reshape(_NW, _CPW, _CHUNK)
    dst = jnp.concatenate([dst, _N + r % (_ACC_ROWS - _N)])
    dst = dst.reshape(_NW, _CPW, _CHUNK)
    ones = jnp.ones((_CHUNK,), jnp.float32)
    zra = jnp.zeros((_CHUNK, _D), jnp.float32)
    zrc = jnp.zeros((_CHUNK,), jnp.float32)
    sums, cnts = _agg(x, src, dst, ones, zra, zrc)
    return _epi(x, sums, cnts)


# speedup vs baseline: 1.0386x; 1.0386x over previous
import jax
import jax.numpy as jnp
from jax import lax
from jax.experimental import pallas as pl
from jax.experimental.pallas import tpu as pltpu
from jax.experimental.pallas import tpu_sc as plsc

_W = 0.5
_N = 10000
_D = 128
_E = 320000
_NC = 2
_NS = 16
_NW = _NC * _NS
_CHUNK = 128
_CPW = 80
_EPAD = _NW * _CPW * _CHUNK
_ACC_ROWS = 10240
_RPS = _ACC_ROWS // _NS


def _agg_body(x_hbm, src_hbm, dst_hbm, ones_hbm, zra_hbm, zrc_hbm,
              sum_hbm, cnt_hbm,
              src_v, dst_a, dst_b, rows_a, rows_b, ones_v, zc_v,
              acc_sh, cnt_sh, sem_a, sem_b, sem_ia, sem_ib, sem_p,
              sem_sa, sem_sb, sem_ca, sem_cb):
    c = lax.axis_index("c")
    s = lax.axis_index("s")
    w = s * _NC + c

    pltpu.sync_copy(zra_hbm, rows_a)
    pltpu.sync_copy(zrc_hbm, zc_v)
    pltpu.sync_copy(ones_hbm, ones_v)
    r0 = s * _RPS
    for k in range(5):
        pltpu.async_copy(rows_a, acc_sh.at[pl.ds(r0 + k * 128, 128)], sem_p)
        pltpu.async_copy(zc_v, cnt_sh.at[pl.ds(r0 + k * 128, 128)], sem_p)
    for k in range(5):
        pltpu.make_async_copy(rows_a, acc_sh.at[pl.ds(r0, 128)], sem_p).wait()
        pltpu.make_async_copy(zc_v, cnt_sh.at[pl.ds(r0, 128)], sem_p).wait()
    plsc.subcore_barrier()

    def gather_start(i, rows_v, sem):
        pltpu.async_copy(x_hbm.at[src_v.at[i]], rows_v, sem)

    def gather_wait(i, rows_v, sem):
        pltpu.make_async_copy(x_hbm.at[src_v.at[i]], rows_v, sem).wait()

    def dst_load(i, dst_c, sem):
        pltpu.async_copy(dst_hbm.at[w, i], dst_c, sem)

    def dst_wait(i, dst_c, sem):
        pltpu.make_async_copy(dst_hbm.at[w, i], dst_c, sem).wait()

    def scatter_start(rows_v, dst_c, sem_s, sem_c):
        pltpu.async_copy(rows_v, acc_sh.at[dst_c], sem_s, add=True)
        pltpu.async_copy(ones_v, cnt_sh.at[dst_c], sem_c, add=True)

    def scatter_wait(rows_v, dst_c, sem_s, sem_c):
        pltpu.make_async_copy(rows_v, acc_sh.at[dst_c], sem_s).wait()
        pltpu.make_async_copy(ones_v, cnt_sh.at[dst_c], sem_c).wait()

    pltpu.sync_copy(src_hbm.at[w], src_v)
    dst_load(0, dst_a, sem_ia)
    dst_load(1, dst_b, sem_ib)
    gather_start(0, rows_a, sem_a)
    gather_start(1, rows_b, sem_b)

    def body(t, carry):
        i = 2 * t
        gather_wait(i, rows_a, sem_a)
        dst_wait(i, dst_a, sem_ia)
        scatter_start(rows_a, dst_a, sem_sa, sem_ca)
        gather_wait(i + 1, rows_b, sem_b)
        dst_wait(i + 1, dst_b, sem_ib)
        scatter_start(rows_b, dst_b, sem_sb, sem_cb)
        scatter_wait(rows_a, dst_a, sem_sa, sem_ca)
        dst_load(i + 2, dst_a, sem_ia)
        gather_start(i + 2, rows_a, sem_a)
        scatter_wait(rows_b, dst_b, sem_sb, sem_cb)
        dst_load(i + 3, dst_b, sem_ib)
        gather_start(i + 3, rows_b, sem_b)
        return carry
    lax.fori_loop(0, _CPW // 2 - 1, body, 0)

    i_last = _CPW - 2
    gather_wait(i_last, rows_a, sem_a)
    dst_wait(i_last, dst_a, sem_ia)
    scatter_start(rows_a, dst_a, sem_sa, sem_ca)
    gather_wait(i_last + 1, rows_b, sem_b)
    dst_wait(i_last + 1, dst_b, sem_ib)
    scatter_start(rows_b, dst_b, sem_sb, sem_cb)
    scatter_wait(rows_a, dst_a, sem_sa, sem_ca)
    scatter_wait(rows_b, dst_b, sem_sb, sem_cb)

    plsc.subcore_barrier()

    for k in range(5):
        pltpu.async_copy(acc_sh.at[pl.ds(r0 + k * 128, 128)],
                         sum_hbm.at[c, pl.ds(r0 + k * 128, 128)], sem_p)
        pltpu.async_copy(cnt_sh.at[pl.ds(r0 + k * 128, 128)],
                         cnt_hbm.at[c, pl.ds(r0 + k * 128, 128)], sem_p)
    for k in range(5):
        pltpu.make_async_copy(acc_sh.at[pl.ds(r0, 128)],
                              sum_hbm.at[c, pl.ds(r0, 128)], sem_p).wait()
        pltpu.make_async_copy(cnt_sh.at[pl.ds(r0, 128)],
                              cnt_hbm.at[c, pl.ds(r0, 128)], sem_p).wait()


_agg = pl.kernel(
    _agg_body,
    mesh=plsc.VectorSubcoreMesh(core_axis_name="c", subcore_axis_name="s"),
    out_type=[
        jax.ShapeDtypeStruct((_NC, _ACC_ROWS, _D), jnp.float32),
        jax.ShapeDtypeStruct((_NC, _ACC_ROWS), jnp.float32),
    ],
    scratch_types=[
        pltpu.VMEM((_CPW, _CHUNK), jnp.int32),
        pltpu.VMEM((_CHUNK,), jnp.int32),
        pltpu.VMEM((_CHUNK,), jnp.int32),
        pltpu.VMEM((_CHUNK, _D), jnp.float32),
        pltpu.VMEM((_CHUNK, _D), jnp.float32),
        pltpu.VMEM((_CHUNK,), jnp.float32),
        pltpu.VMEM((_CHUNK,), jnp.float32),
        pltpu.VMEM_SHARED((_ACC_ROWS, _D), jnp.float32),
        pltpu.VMEM_SHARED((_ACC_ROWS,), jnp.float32),
        pltpu.SemaphoreType.DMA,
        pltpu.SemaphoreType.DMA,
        pltpu.SemaphoreType.DMA,
        pltpu.SemaphoreType.DMA,
        pltpu.SemaphoreType.DMA,
        pltpu.SemaphoreType.DMA,
        pltpu.SemaphoreType.DMA,
        pltpu.SemaphoreType.DMA,
        pltpu.SemaphoreType.DMA,
    ],
)


def _epi_body(x_ref, s_ref, c_ref, o_ref):
    cnt = c_ref[0, 0:_N] + c_ref[1, 0:_N]
    cnt = jnp.maximum(cnt, 1.0).reshape(_N, 1)
    mean = (s_ref[0, 0:_N] + s_ref[1, 0:_N]) / cnt
    o_ref[...] = x_ref[...] + _W * mean


_epi = pl.pallas_call(
    _epi_body,
    out_shape=jax.ShapeDtypeStruct((_N, _D), jnp.float32),
)


def kernel(x, edge_index):
    src = edge_index[0].astype(jnp.int32)
    dst = edge_index[1].astype(jnp.int32)
    pad = _EPAD - _E
    r = jnp.arange(pad, dtype=jnp.int32)
    src = jnp.concatenate([src, r % _N]).reshape(_NW, _CPW, _CHUNK)
    dst = jnp.concatenate([dst, _N + r % (_ACC_ROWS - _N)])
    dst = dst.reshape(_NW, _CPW, _CHUNK)
    ones = jnp.ones((_CHUNK,), jnp.float32)
    zra = jnp.zeros((_CHUNK, _D), jnp.float32)
    zrc = jnp.zeros((_CHUNK,), jnp.float32)
    sums, cnts = _agg(x, src, dst, ones, zra, zrc)
    return _epi(x, sums, cnts)


# speedup vs baseline: 1.2515x; 1.2051x over previous
import jax
import jax.numpy as jnp
from jax import lax
from jax.experimental import pallas as pl
from jax.experimental.pallas import tpu as pltpu
from jax.experimental.pallas import tpu_sc as plsc

_W = 0.5
_N = 10000
_D = 128
_E = 320000
_NC = 2
_NS = 16
_NW = _NC * _NS
_CHUNK = 128
_CPW = 80
_EPAD = _NW * _CPW * _CHUNK
_ACC_ROWS = 10240


def _agg_body(x_hbm, ep_hbm, ones_hbm, zra_hbm, zrc_hbm,
              sum_hbm, cnt_hbm,
              ep_v, src_a, dst_a, src_b, dst_b, rows_a, rows_b, ones_v, zc_v,
              acc_sh, cnt_sh, sem_a, sem_b, sem_p):
    c = lax.axis_index("c")
    s = lax.axis_index("s")
    w = s * _NC + c

    pltpu.sync_copy(zra_hbm, rows_a)
    pltpu.sync_copy(zrc_hbm, zc_v)
    r0 = s * 640
    for k in range(5):
        pltpu.async_copy(rows_a, acc_sh.at[pl.ds(r0 + k * 128, 128)], sem_p)
        pltpu.async_copy(zc_v, cnt_sh.at[pl.ds(r0 + k * 128, 128)], sem_p)
    pltpu.sync_copy(ep_hbm.at[w], ep_v)
    pltpu.sync_copy(ones_hbm, ones_v)
    for k in range(5):
        pltpu.make_async_copy(rows_a, acc_sh.at[pl.ds(r0, 128)], sem_p).wait()
        pltpu.make_async_copy(zc_v, cnt_sh.at[pl.ds(r0, 128)], sem_p).wait()
    plsc.subcore_barrier()

    def unpack(i, src_c, dst_c):
        for j in range(_CHUNK // 16):
            p = ep_v[i, pl.ds(j * 16, 16)]
            src_c[pl.ds(j * 16, 16)] = jnp.bitwise_and(p, 16383)
            dst_c[pl.ds(j * 16, 16)] = jnp.right_shift(p, 14)

    def consume(src_c, rows_v, dst_c, sem):
        pltpu.make_async_copy(x_hbm.at[src_c], rows_v, sem).wait()
        pltpu.sync_copy(rows_v, acc_sh.at[dst_c], add=True)
        pltpu.sync_copy(ones_v, cnt_sh.at[dst_c], add=True)

    unpack(0, src_a, dst_a)
    pltpu.async_copy(x_hbm.at[src_a], rows_a, sem_a)

    def body(t, carry):
        i = 2 * t
        unpack(i + 1, src_b, dst_b)
        pltpu.async_copy(x_hbm.at[src_b], rows_b, sem_b)
        consume(src_a, rows_a, dst_a, sem_a)
        unpack(i + 2, src_a, dst_a)
        pltpu.async_copy(x_hbm.at[src_a], rows_a, sem_a)
        consume(src_b, rows_b, dst_b, sem_b)
        return carry
    lax.fori_loop(0, _CPW // 2 - 1, body, 0)

    unpack(_CPW - 1, src_b, dst_b)
    pltpu.async_copy(x_hbm.at[src_b], rows_b, sem_b)
    consume(src_a, rows_a, dst_a, sem_a)
    consume(src_b, rows_b, dst_b, sem_b)

    plsc.subcore_barrier()

    for k in range(5):
        pltpu.async_copy(acc_sh.at[pl.ds(r0 + k * 128, 128)],
                         sum_hbm.at[c, pl.ds(r0 + k * 128, 128)], sem_p)
        pltpu.async_copy(cnt_sh.at[pl.ds(r0 + k * 128, 128)],
                         cnt_hbm.at[c, pl.ds(r0 + k * 128, 128)], sem_p)
    for k in range(5):
        pltpu.make_async_copy(acc_sh.at[pl.ds(r0, 128)],
                              sum_hbm.at[c, pl.ds(r0, 128)], sem_p).wait()
        pltpu.make_async_copy(cnt_sh.at[pl.ds(r0, 128)],
                              cnt_hbm.at[c, pl.ds(r0, 128)], sem_p).wait()


_agg = pl.kernel(
    _agg_body,
    mesh=plsc.VectorSubcoreMesh(core_axis_name="c", subcore_axis_name="s"),
    out_type=[
        jax.ShapeDtypeStruct((_NC, _ACC_ROWS, _D), jnp.float32),
        jax.ShapeDtypeStruct((_NC, _ACC_ROWS), jnp.float32),
    ],
    scratch_types=[
        pltpu.VMEM((_CPW, _CHUNK), jnp.int32),
        pltpu.VMEM((_CHUNK,), jnp.int32),
        pltpu.VMEM((_CHUNK,), jnp.int32),
        pltpu.VMEM((_CHUNK,), jnp.int32),
        pltpu.VMEM((_CHUNK,), jnp.int32),
        pltpu.VMEM((_CHUNK, _D), jnp.float32),
        pltpu.VMEM((_CHUNK, _D), jnp.float32),
        pltpu.VMEM((_CHUNK,), jnp.float32),
        pltpu.VMEM((_CHUNK,), jnp.float32),
        pltpu.VMEM_SHARED((_ACC_ROWS, _D), jnp.float32),
        pltpu.VMEM_SHARED((_ACC_ROWS,), jnp.float32),
        pltpu.SemaphoreType.DMA,
        pltpu.SemaphoreType.DMA,
        pltpu.SemaphoreType.DMA,
    ],
)


_NCH = _EPAD // _CHUNK
_MCH = _E // _CHUNK


def _pack_body(e_ref, o_ref):
    src = e_ref[0]
    dst = e_ref[1]
    main = jnp.left_shift(dst, 14) | src
    o_ref[0:_MCH] = main
    rows = jax.lax.broadcasted_iota(jnp.int32, (_NCH - _MCH, _CHUNK), 0)
    lanes = jax.lax.broadcasted_iota(jnp.int32, (_NCH - _MCH, _CHUNK), 1)
    flat = rows * _CHUNK + lanes
    o_ref[_MCH:_NCH] = (
        jnp.left_shift(_N + flat % (_ACC_ROWS - _N), 14) | (flat % _N))


_pack = pl.pallas_call(
    _pack_body,
    out_shape=jax.ShapeDtypeStruct((_NCH, _CHUNK), jnp.int32),
)


def _epi_body(x_ref, s_ref, c_ref, o_ref):
    cnt = c_ref[0, 0:_N] + c_ref[1, 0:_N]
    cnt = jnp.maximum(cnt, 1.0).reshape(_N, 1)
    mean = (s_ref[0, 0:_N] + s_ref[1, 0:_N]) / cnt
    o_ref[...] = x_ref[...] + _W * mean


_epi = pl.pallas_call(
    _epi_body,
    out_shape=jax.ShapeDtypeStruct((_N, _D), jnp.float32),
)


def kernel(x, edge_index):
    e3 = edge_index.astype(jnp.int32).reshape(2, _MCH, _CHUNK)
    packed = _pack(e3).reshape(_NW, _CPW, _CHUNK)
    ones = jnp.ones((_CHUNK,), jnp.float32)
    zra = jnp.zeros((_CHUNK, _D), jnp.float32)
    zrc = jnp.zeros((_CHUNK,), jnp.float32)
    sums, cnts = _agg(x, packed, ones, zra, zrc)
    return _epi(x, sums, cnts)


# speedup vs baseline: 1.2986x; 1.0376x over previous
import jax
import jax.numpy as jnp
from jax import lax
from jax.experimental import pallas as pl
from jax.experimental.pallas import tpu as pltpu
from jax.experimental.pallas import tpu_sc as plsc

_W = 0.5
_N = 10000
_D = 128
_E = 320000
_NC = 2
_NS = 16
_NW = _NC * _NS
_CHUNK = 128
_CPW = 80
_EPAD = _NW * _CPW * _CHUNK
_ACC_ROWS = 10240


def _agg_body(x_hbm, ep_hbm, ones_hbm, zra_hbm, zrc_hbm,
              sum_hbm, cnt_hbm,
              ep_v, src_a, dst_a, src_b, dst_b, rows_a, rows_b, ones_v, zc_v,
              acc_sh, cnt_sh, sem_a, sem_b, sem_p):
    c = lax.axis_index("c")
    s = lax.axis_index("s")
    w = s * _NC + c

    pltpu.sync_copy(zra_hbm, rows_a)
    pltpu.sync_copy(zrc_hbm, zc_v)
    r0 = s * 640
    for k in range(5):
        pltpu.async_copy(rows_a, acc_sh.at[pl.ds(r0 + k * 128, 128)], sem_p)
        pltpu.async_copy(zc_v, cnt_sh.at[pl.ds(r0 + k * 128, 128)], sem_p)
    pltpu.sync_copy(ep_hbm.at[pl.ds(w * _CPW, _CPW)], ep_v)
    pltpu.sync_copy(ones_hbm, ones_v)
    for k in range(5):
        pltpu.make_async_copy(rows_a, acc_sh.at[pl.ds(r0, 128)], sem_p).wait()
        pltpu.make_async_copy(zc_v, cnt_sh.at[pl.ds(r0, 128)], sem_p).wait()
    plsc.subcore_barrier()

    def unpack(i, src_c, dst_c):
        for j in range(_CHUNK // 16):
            p = ep_v[i, pl.ds(j * 16, 16)]
            src_c[pl.ds(j * 16, 16)] = jnp.bitwise_and(p, 16383)
            dst_c[pl.ds(j * 16, 16)] = jnp.right_shift(p, 14)

    def consume(src_c, rows_v, dst_c, sem):
        pltpu.make_async_copy(x_hbm.at[src_c], rows_v, sem).wait()
        pltpu.sync_copy(rows_v, acc_sh.at[dst_c], add=True)
        pltpu.sync_copy(ones_v, cnt_sh.at[dst_c], add=True)

    unpack(0, src_a, dst_a)
    pltpu.async_copy(x_hbm.at[src_a], rows_a, sem_a)

    def body(t, carry):
        i = 2 * t
        unpack(i + 1, src_b, dst_b)
        pltpu.async_copy(x_hbm.at[src_b], rows_b, sem_b)
        consume(src_a, rows_a, dst_a, sem_a)
        unpack(i + 2, src_a, dst_a)
        pltpu.async_copy(x_hbm.at[src_a], rows_a, sem_a)
        consume(src_b, rows_b, dst_b, sem_b)
        return carry
    lax.fori_loop(0, _CPW // 2 - 1, body, 0)

    unpack(_CPW - 1, src_b, dst_b)
    pltpu.async_copy(x_hbm.at[src_b], rows_b, sem_b)
    consume(src_a, rows_a, dst_a, sem_a)
    consume(src_b, rows_b, dst_b, sem_b)

    plsc.subcore_barrier()

    for k in range(5):
        pltpu.async_copy(acc_sh.at[pl.ds(r0 + k * 128, 128)],
                         sum_hbm.at[c, pl.ds(r0 + k * 128, 128)], sem_p)
        pltpu.async_copy(cnt_sh.at[pl.ds(r0 + k * 128, 128)],
                         cnt_hbm.at[c, pl.ds(r0 + k * 128, 128)], sem_p)
    for k in range(5):
        pltpu.make_async_copy(acc_sh.at[pl.ds(r0, 128)],
                              sum_hbm.at[c, pl.ds(r0, 128)], sem_p).wait()
        pltpu.make_async_copy(cnt_sh.at[pl.ds(r0, 128)],
                              cnt_hbm.at[c, pl.ds(r0, 128)], sem_p).wait()


_agg = pl.kernel(
    _agg_body,
    mesh=plsc.VectorSubcoreMesh(core_axis_name="c", subcore_axis_name="s"),
    out_type=[
        jax.ShapeDtypeStruct((_NC, _ACC_ROWS, _D), jnp.float32),
        jax.ShapeDtypeStruct((_NC, _ACC_ROWS), jnp.float32),
    ],
    scratch_types=[
        pltpu.VMEM((_CPW, _CHUNK), jnp.int32),
        pltpu.VMEM((_CHUNK,), jnp.int32),
        pltpu.VMEM((_CHUNK,), jnp.int32),
        pltpu.VMEM((_CHUNK,), jnp.int32),
        pltpu.VMEM((_CHUNK,), jnp.int32),
        pltpu.VMEM((_CHUNK, _D), jnp.float32),
        pltpu.VMEM((_CHUNK, _D), jnp.float32),
        pltpu.VMEM((_CHUNK,), jnp.float32),
        pltpu.VMEM((_CHUNK,), jnp.float32),
        pltpu.VMEM_SHARED((_ACC_ROWS, _D), jnp.float32),
        pltpu.VMEM_SHARED((_ACC_ROWS,), jnp.float32),
        pltpu.SemaphoreType.DMA,
        pltpu.SemaphoreType.DMA,
        pltpu.SemaphoreType.DMA,
    ],
)


_NCH = _EPAD // _CHUNK
_MCH = _E // _CHUNK


def _pack_body(e_ref, o_ref):
    src = e_ref[0].reshape(_MCH, _CHUNK)
    dst = e_ref[1].reshape(_MCH, _CHUNK)
    main = jnp.left_shift(dst, 14) | src
    o_ref[0:_MCH] = main
    rows = jax.lax.broadcasted_iota(jnp.int32, (_NCH - _MCH, _CHUNK), 0)
    lanes = jax.lax.broadcasted_iota(jnp.int32, (_NCH - _MCH, _CHUNK), 1)
    flat = rows * _CHUNK + lanes
    o_ref[_MCH:_NCH] = (
        jnp.left_shift(_N + flat % (_ACC_ROWS - _N), 14) | (flat % _N))


_pack = pl.pallas_call(
    _pack_body,
    out_shape=jax.ShapeDtypeStruct((_NCH, _CHUNK), jnp.int32),
)


def _epi_body(x_ref, s_ref, c_ref, o_ref):
    cnt = c_ref[0, 0:_N] + c_ref[1, 0:_N]
    cnt = jnp.maximum(cnt, 1.0).reshape(_N, 1)
    mean = (s_ref[0, 0:_N] + s_ref[1, 0:_N]) / cnt
    o_ref[...] = x_ref[...] + _W * mean


_epi = pl.pallas_call(
    _epi_body,
    out_shape=jax.ShapeDtypeStruct((_N, _D), jnp.float32),
)


def kernel(x, edge_index):
    packed = _pack(edge_index.astype(jnp.int32))
    ones = jnp.ones((_CHUNK,), jnp.float32)
    zra = jnp.zeros((_CHUNK, _D), jnp.float32)
    zrc = jnp.zeros((_CHUNK,), jnp.float32)
    sums, cnts = _agg(x, packed, ones, zra, zrc)
    return _epi(x, sums, cnts)


# speedup vs baseline: 1.3006x; 1.0015x over previous
import jax
import jax.numpy as jnp
import numpy as np
from jax import lax
from jax.experimental import pallas as pl
from jax.experimental.pallas import tpu as pltpu
from jax.experimental.pallas import tpu_sc as plsc

_W = 0.5
_N = 10000
_D = 128
_E = 320000
_NC = 2
_NS = 16
_NW = _NC * _NS
_CHUNK = 128
_CPW = 80
_EPAD = _NW * _CPW * _CHUNK
_ACC_ROWS = 10240


def _agg_body(x_hbm, ep_hbm, ones_hbm, zra_hbm, zrc_hbm,
              sum_hbm, cnt_hbm,
              ep_v, src_a, dst_a, src_b, dst_b, rows_a, rows_b, ones_v, zc_v,
              acc_sh, cnt_sh, sem_a, sem_b, sem_p):
    c = lax.axis_index("c")
    s = lax.axis_index("s")
    w = s * _NC + c

    pltpu.sync_copy(zra_hbm, rows_a)
    pltpu.sync_copy(zrc_hbm, zc_v)
    r0 = s * 640
    for k in range(5):
        pltpu.async_copy(rows_a, acc_sh.at[pl.ds(r0 + k * 128, 128)], sem_p)
        pltpu.async_copy(zc_v, cnt_sh.at[pl.ds(r0 + k * 128, 128)], sem_p)
    pltpu.sync_copy(ep_hbm.at[pl.ds(w * _CPW, _CPW)], ep_v)
    pltpu.sync_copy(ones_hbm, ones_v)
    for k in range(5):
        pltpu.make_async_copy(rows_a, acc_sh.at[pl.ds(r0, 128)], sem_p).wait()
        pltpu.make_async_copy(zc_v, cnt_sh.at[pl.ds(r0, 128)], sem_p).wait()
    plsc.subcore_barrier()

    def unpack(i, src_c, dst_c):
        for j in range(_CHUNK // 16):
            p = ep_v[i, pl.ds(j * 16, 16)]
            src_c[pl.ds(j * 16, 16)] = jnp.bitwise_and(p, 16383)
            dst_c[pl.ds(j * 16, 16)] = jnp.right_shift(p, 14)

    def consume(src_c, rows_v, dst_c, sem):
        pltpu.make_async_copy(x_hbm.at[src_c], rows_v, sem).wait()
        pltpu.sync_copy(rows_v, acc_sh.at[dst_c], add=True)
        pltpu.sync_copy(ones_v, cnt_sh.at[dst_c], add=True)

    unpack(0, src_a, dst_a)
    pltpu.async_copy(x_hbm.at[src_a], rows_a, sem_a)

    def body(t, carry):
        i = 2 * t
        unpack(i + 1, src_b, dst_b)
        pltpu.async_copy(x_hbm.at[src_b], rows_b, sem_b)
        consume(src_a, rows_a, dst_a, sem_a)
        unpack(i + 2, src_a, dst_a)
        pltpu.async_copy(x_hbm.at[src_a], rows_a, sem_a)
        consume(src_b, rows_b, dst_b, sem_b)
        return carry
    lax.fori_loop(0, _CPW // 2 - 1, body, 0)

    unpack(_CPW - 1, src_b, dst_b)
    pltpu.async_copy(x_hbm.at[src_b], rows_b, sem_b)
    consume(src_a, rows_a, dst_a, sem_a)
    consume(src_b, rows_b, dst_b, sem_b)

    plsc.subcore_barrier()

    for k in range(5):
        pltpu.async_copy(acc_sh.at[pl.ds(r0 + k * 128, 128)],
                         sum_hbm.at[c, pl.ds(r0 + k * 128, 128)], sem_p)
        pltpu.async_copy(cnt_sh.at[pl.ds(r0 + k * 128, 128)],
                         cnt_hbm.at[c, pl.ds(r0 + k * 128, 128)], sem_p)
    for k in range(5):
        pltpu.make_async_copy(acc_sh.at[pl.ds(r0, 128)],
                              sum_hbm.at[c, pl.ds(r0, 128)], sem_p).wait()
        pltpu.make_async_copy(cnt_sh.at[pl.ds(r0, 128)],
                              cnt_hbm.at[c, pl.ds(r0, 128)], sem_p).wait()


_agg = pl.kernel(
    _agg_body,
    mesh=plsc.VectorSubcoreMesh(core_axis_name="c", subcore_axis_name="s"),
    out_type=[
        jax.ShapeDtypeStruct((_NC, _ACC_ROWS, _D), jnp.float32),
        jax.ShapeDtypeStruct((_NC, _ACC_ROWS), jnp.float32),
    ],
    scratch_types=[
        pltpu.VMEM((_CPW, _CHUNK), jnp.int32),
        pltpu.VMEM((_CHUNK,), jnp.int32),
        pltpu.VMEM((_CHUNK,), jnp.int32),
        pltpu.VMEM((_CHUNK,), jnp.int32),
        pltpu.VMEM((_CHUNK,), jnp.int32),
        pltpu.VMEM((_CHUNK, _D), jnp.float32),
        pltpu.VMEM((_CHUNK, _D), jnp.float32),
        pltpu.VMEM((_CHUNK,), jnp.float32),
        pltpu.VMEM((_CHUNK,), jnp.float32),
        pltpu.VMEM_SHARED((_ACC_ROWS, _D), jnp.float32),
        pltpu.VMEM_SHARED((_ACC_ROWS,), jnp.float32),
        pltpu.SemaphoreType.DMA,
        pltpu.SemaphoreType.DMA,
        pltpu.SemaphoreType.DMA,
    ],
)


_NCH = _EPAD // _CHUNK
_MCH = _E // _CHUNK


def _pack_body(e_ref, o_ref):
    src = e_ref[0].reshape(_MCH, _CHUNK)
    dst = e_ref[1].reshape(_MCH, _CHUNK)
    main = jnp.left_shift(dst, 14) | src
    o_ref[0:_MCH] = main
    rows = jax.lax.broadcasted_iota(jnp.int32, (_NCH - _MCH, _CHUNK), 0)
    lanes = jax.lax.broadcasted_iota(jnp.int32, (_NCH - _MCH, _CHUNK), 1)
    flat = rows * _CHUNK + lanes
    o_ref[_MCH:_NCH] = (
        jnp.left_shift(_N + flat % (_ACC_ROWS - _N), 14) | (flat % _N))


_pack = pl.pallas_call(
    _pack_body,
    out_shape=jax.ShapeDtypeStruct((_NCH, _CHUNK), jnp.int32),
)


def _epi_body(x_ref, s_ref, c_ref, o_ref):
    cnt = c_ref[0, 0:_N] + c_ref[1, 0:_N]
    cnt = jnp.maximum(cnt, 1.0).reshape(_N, 1)
    mean = (s_ref[0, 0:_N] + s_ref[1, 0:_N]) / cnt
    o_ref[...] = x_ref[...] + _W * mean


_epi = pl.pallas_call(
    _epi_body,
    out_shape=jax.ShapeDtypeStruct((_N, _D), jnp.float32),
)


def kernel(x, edge_index):
    packed = _pack(edge_index.astype(jnp.int32))
    ones = np.ones((_CHUNK,), np.float32)
    zra = np.zeros((_CHUNK, _D), np.float32)
    zrc = np.zeros((_CHUNK,), np.float32)
    sums, cnts = _agg(x, packed, ones, zra, zrc)
    return _epi(x, sums, cnts)
